# baseline (speedup 1.0000x reference)
import functools

import jax
import jax.numpy as jnp
from jax import lax
from jax.experimental import pallas as pl
from jax.experimental.pallas import tpu as pltpu
from jax.experimental.pallas import tpu_sc as plsc

B = 16384
D = 64

NC = 2
NS = 16
NW = NC * NS
BPW = B // NW
IDX_CHUNK = 128
NCHUNK = BPW // IDX_CHUNK

BLK = 2048


def _gather_body(uidx_hbm, iidx_hbm, utab_hbm, itab_hbm, u_out, i_out,
                 uidx_v, iidx_v, urows_v, irows_v, sem):
  wid = lax.axis_index("s") * NC + lax.axis_index("c")
  base = wid * BPW
  pltpu.sync_copy(uidx_hbm.at[pl.ds(base, BPW)], uidx_v)
  pltpu.sync_copy(iidx_hbm.at[pl.ds(base, BPW)], iidx_v)
  copies = []
  for j in range(NCHUNK):
    sl = pl.ds(j * IDX_CHUNK, IDX_CHUNK)
    copies.append(pltpu.async_copy(utab_hbm.at[uidx_v.at[sl]], urows_v.at[sl], sem))
    copies.append(pltpu.async_copy(itab_hbm.at[iidx_v.at[sl]], irows_v.at[sl], sem))
  for c in copies:
    c.wait()
  pltpu.sync_copy(urows_v, u_out.at[pl.ds(base, BPW)])
  pltpu.sync_copy(irows_v, i_out.at[pl.ds(base, BPW)])


def _sc_gather(user_idx, item_idx, user_table, item_table):
  mesh = plsc.VectorSubcoreMesh(core_axis_name="c", subcore_axis_name="s")
  f = pl.kernel(
      _gather_body,
      out_type=(
          jax.ShapeDtypeStruct((B, D), jnp.float32),
          jax.ShapeDtypeStruct((B, D), jnp.float32),
      ),
      mesh=mesh,
      scratch_types=[
          pltpu.VMEM((BPW,), jnp.int32),
          pltpu.VMEM((BPW,), jnp.int32),
          pltpu.VMEM((BPW, D), jnp.float32),
          pltpu.VMEM((BPW, D), jnp.float32),
          pltpu.SemaphoreType.DMA,
      ],
      compiler_params=pltpu.CompilerParams(use_tc_tiling_on_sc=False),
  )
  return f(user_idx, item_idx, user_table, item_table)


def _mlp_body(u_ref, i_ref, wa_ref, wb_ref, wc_ref, b1_ref, w2t_ref, b2_ref, o_ref):
  u = u_ref[...]
  v = i_ref[...]
  e = u * v
  h = (jnp.dot(e, wa_ref[...], preferred_element_type=jnp.float32)
       + jnp.dot(u, wb_ref[...], preferred_element_type=jnp.float32)
       + jnp.dot(v, wc_ref[...], preferred_element_type=jnp.float32)
       + b1_ref[...])
  h = jnp.maximum(h, 0.0)
  z = jnp.sum(h * w2t_ref[...], axis=1, keepdims=True) + b2_ref[...]
  o_ref[...] = jax.nn.sigmoid(z)


def _tc_mlp(u, i, W1, b1, W2, b2):
  wa = W1[0:D]
  wb = W1[D:2 * D]
  wc = W1[2 * D:3 * D]
  b1r = b1.reshape(1, 8)
  w2t = W2.reshape(1, 8)
  b2r = b2.reshape(1, 1)
  grid = (B // BLK,)
  return pl.pallas_call(
      _mlp_body,
      grid=grid,
      in_specs=[
          pl.BlockSpec((BLK, D), lambda n: (n, 0)),
          pl.BlockSpec((BLK, D), lambda n: (n, 0)),
          pl.BlockSpec((D, 8), lambda n: (0, 0)),
          pl.BlockSpec((D, 8), lambda n: (0, 0)),
          pl.BlockSpec((D, 8), lambda n: (0, 0)),
          pl.BlockSpec((1, 8), lambda n: (0, 0)),
          pl.BlockSpec((1, 8), lambda n: (0, 0)),
          pl.BlockSpec((1, 1), lambda n: (0, 0)),
      ],
      out_specs=pl.BlockSpec((BLK, 1), lambda n: (n, 0)),
      out_shape=jax.ShapeDtypeStruct((B, 1), jnp.float32),
      compiler_params=pltpu.CompilerParams(
          dimension_semantics=("arbitrary",),
      ),
  )(u, i, wa, wb, wc, b1r, w2t, b2r)


@jax.jit
def kernel(group_inputs, user_inputs, item_inputs, user_table, item_table, W1, b1, W2, b2):
  del group_inputs
  u, i = _sc_gather(user_inputs.astype(jnp.int32), item_inputs.astype(jnp.int32),
                    user_table, item_table)
  return _tc_mlp(u, i, W1, b1, W2, b2)
